# Initial kernel scaffold; baseline (speedup 1.0000x reference)
import jax
import jax.numpy as jnp
from jax.experimental import pallas as pl


def kernel(x, edge_attr, node_batch, W, b):
    raise NotImplementedError("write your pallas kernel here")


# speedup vs baseline: 2.0969x; 2.0969x over previous
import functools

import jax
import jax.numpy as jnp
from jax import lax
from jax.experimental import pallas as pl
from jax.experimental.pallas import tpu as pltpu
from jax.experimental.pallas import tpu_sc as plsc

N = 100000
NHID = 128
G = 512
L = 16
NC = 2
NS = 16
NW = NC * NS
ROWS_W = 3128
RB = 136
NBLK = ROWS_W // RB
IDS_PAD = NW * ROWS_W

_mesh = plsc.VectorSubcoreMesh(
    core_axis_name="c", subcore_axis_name="s", num_cores=NC, num_subcores=NS
)


@functools.partial(
    pl.kernel,
    out_type=jax.ShapeDtypeStruct((NW, G * NHID), jnp.float32),
    mesh=_mesh,
    compiler_params=pltpu.CompilerParams(needs_layout_passes=False),
    scratch_types=[
        pltpu.VMEM((ROWS_W,), jnp.int32),
        pltpu.VMEM((RB * NHID,), jnp.float32),
        pltpu.VMEM((G * NHID,), jnp.float32),
    ],
)
def _sc_segment_sum(ids_hbm, x_hbm, out_hbm, ids_v, xbuf, acc):
    wid = lax.axis_index("s") * NC + lax.axis_index("c")
    row0 = pl.multiple_of(wid * ROWS_W, 8)
    row_end = jnp.minimum(row0 + ROWS_W, N)

    iota = lax.broadcasted_iota(jnp.int32, (L,), 0)
    cols = [iota + L * j for j in range(NHID // L)]
    zero16 = jnp.zeros((L,), jnp.float32)

    pltpu.sync_copy(ids_hbm.at[pl.ds(row0, ROWS_W)], ids_v)

    def _zero(g, carry):
        g16 = jnp.full((L,), g * NHID, jnp.int32)
        for j in range(NHID // L):
            plsc.store_scatter(acc, [g16 + cols[j]], zero16)
        return carry

    lax.fori_loop(0, G, _zero, 0)

    def _block(b, carry):
        base = row0 + b * RB
        base_c = jnp.minimum(base, N - RB)
        src0 = pl.multiple_of(base_c * NHID, 8)
        pltpu.sync_copy(x_hbm.at[pl.ds(src0, RB * NHID)], xbuf)

        def _row(r, c2):
            gr = base_c + r
            valid = jnp.logical_and(gr >= base, gr < row_end)
            m16 = jnp.full((L,), valid)
            seg16 = plsc.load_gather(ids_v, [jnp.full((L,), gr - row0, jnp.int32)])
            seg_base = seg16 * NHID
            r16 = jnp.full((L,), r * NHID, jnp.int32)
            for j in range(NHID // L):
                xv = plsc.load_gather(xbuf, [r16 + cols[j]])
                plsc.addupdate_scatter(acc, [seg_base + cols[j]], xv, mask=m16)
            return c2

        lax.fori_loop(0, RB, _row, 0)
        return carry

    lax.fori_loop(0, NBLK, _block, 0)

    pltpu.sync_copy(acc, out_hbm.at[wid])


def _tc_finish_body(p_ref, w_ref, b_ref, o_ref):
    pooled = jnp.sum(p_ref[...].reshape(NW, G, NHID), axis=0)
    o_ref[...] = (
        lax.dot_general(
            pooled,
            w_ref[...],
            (((1,), (1,)), ((), ())),
            preferred_element_type=jnp.float32,
        )
        + b_ref[...]
    )


def kernel(x, edge_attr, node_batch, W, b):
    del edge_attr
    ids = jnp.zeros((IDS_PAD,), jnp.int32).at[:N].set(node_batch.astype(jnp.int32))
    partials = _sc_segment_sum(ids, x.reshape(N * NHID))
    out = pl.pallas_call(
        _tc_finish_body,
        out_shape=jax.ShapeDtypeStruct((G, NHID), jnp.float32),
    )(partials, W, b.reshape(1, NHID))
    return out


# speedup vs baseline: 4.6197x; 2.2031x over previous
import functools

import jax
import jax.numpy as jnp
from jax import lax
from jax.experimental import pallas as pl
from jax.experimental.pallas import tpu as pltpu
from jax.experimental.pallas import tpu_sc as plsc

N = 100000
NHID = 128
G = 512
NC = 2
NS = 16
NW = NC * NS
RB = 128
NFULL = N // RB
TAIL = N - NFULL * RB
TAIL_BASE = N - RB
NBLOCKS = NFULL + 1
BASE_BLK = NBLOCKS // NW
EXTRA = NBLOCKS - BASE_BLK * NW
PAD_SEG = G
ACC_ROWS = 640
IDS_ROWS = 784
ZROWS = ACC_ROWS // NS

_mesh = plsc.VectorSubcoreMesh(
    core_axis_name="c", subcore_axis_name="s", num_cores=NC, num_subcores=NS
)


@functools.partial(
    pl.kernel,
    out_type=jax.ShapeDtypeStruct((NC, G, NHID), jnp.float32),
    mesh=_mesh,
    compiler_params=pltpu.CompilerParams(needs_layout_passes=False),
    scratch_types=[
        pltpu.VMEM((32, RB), jnp.int32),
        pltpu.VMEM((RB, NHID), jnp.float32),
        pltpu.VMEM((ZROWS, NHID), jnp.float32),
        pltpu.VMEM_SHARED((ACC_ROWS, NHID), jnp.float32),
    ],
)
def _sc_segment_sum(ids_hbm, x_hbm, zeros_hbm, out_hbm, ids_v, xbuf, zbuf, acc_sh):
    cid = lax.axis_index("c")
    sid = lax.axis_index("s")
    wid = sid * NC + cid

    blk0 = wid * BASE_BLK + jnp.minimum(wid, EXTRA)
    nblk = BASE_BLK + jnp.where(wid < EXTRA, 1, 0)

    pltpu.sync_copy(zeros_hbm, zbuf)
    pltpu.sync_copy(zbuf, acc_sh.at[pl.ds(pl.multiple_of(sid * ZROWS, 8), ZROWS)])

    ids0 = pl.multiple_of((blk0 // 8) * 8, 8)
    pltpu.sync_copy(ids_hbm.at[pl.ds(ids0, 32)], ids_v)

    plsc.subcore_barrier()

    def _block(j, carry):
        blk = blk0 + j
        row_base = pl.multiple_of(
            jnp.where(blk == NFULL, TAIL_BASE, blk * RB), 8
        )
        pltpu.sync_copy(x_hbm.at[pl.ds(row_base, RB)], xbuf)
        pltpu.sync_copy(xbuf, acc_sh.at[ids_v.at[blk - ids0]], add=True)
        return carry

    lax.fori_loop(0, nblk, _block, 0)

    plsc.subcore_barrier()

    rows_out = G // NS
    r0 = pl.multiple_of(sid * rows_out, 8)
    pltpu.sync_copy(acc_sh.at[pl.ds(r0, rows_out)], zbuf.at[pl.ds(0, rows_out)])
    pltpu.sync_copy(zbuf.at[pl.ds(0, rows_out)], out_hbm.at[cid].at[pl.ds(r0, rows_out)])


def _tc_finish_body(p_ref, w_ref, b_ref, o_ref):
    pooled = p_ref[0] + p_ref[1]
    o_ref[...] = (
        lax.dot_general(
            pooled,
            w_ref[...],
            (((1,), (1,)), ((), ())),
            preferred_element_type=jnp.float32,
        )
        + b_ref[...]
    )


def kernel(x, edge_attr, node_batch, W, b):
    del edge_attr
    ids = node_batch.astype(jnp.int32)
    tail_ids = jnp.concatenate(
        [jnp.full((RB - TAIL,), PAD_SEG, jnp.int32), ids[NFULL * RB :]]
    )
    ids2d = jnp.concatenate([ids[: NFULL * RB], tail_ids]).reshape(NBLOCKS, RB)
    ids2d = jnp.concatenate(
        [ids2d, jnp.full((IDS_ROWS - NBLOCKS, RB), PAD_SEG, jnp.int32)]
    )
    zeros = jnp.zeros((ZROWS, NHID), jnp.float32)
    partials = _sc_segment_sum(ids2d, x, zeros)
    out = pl.pallas_call(
        _tc_finish_body,
        out_shape=jax.ShapeDtypeStruct((G, NHID), jnp.float32),
    )(partials, W, b.reshape(1, NHID))
    return out


# speedup vs baseline: 6.0697x; 1.3139x over previous
import functools

import jax
import jax.numpy as jnp
from jax import lax
from jax.experimental import pallas as pl
from jax.experimental.pallas import tpu as pltpu
from jax.experimental.pallas import tpu_sc as plsc

N = 100000
NHID = 128
G = 512
NC = 2
NS = 16
NW = NC * NS
RB = 128
NFULL = N // RB
TAIL = N - NFULL * RB
TAIL_BASE = N - RB
NBLOCKS = NFULL + 1
BASE_BLK = NBLOCKS // NW
EXTRA = NBLOCKS - BASE_BLK * NW
PAD_SEG = G
ACC_ROWS = 640
IDS_ROWS = 784
ZROWS = ACC_ROWS // NS

_mesh = plsc.VectorSubcoreMesh(
    core_axis_name="c", subcore_axis_name="s", num_cores=NC, num_subcores=NS
)


@functools.partial(
    pl.kernel,
    out_type=jax.ShapeDtypeStruct((NC, G, NHID), jnp.float32),
    mesh=_mesh,
    compiler_params=pltpu.CompilerParams(needs_layout_passes=False),
    scratch_types=[
        pltpu.VMEM((32, RB), jnp.int32),
        pltpu.VMEM((2, RB, NHID), jnp.float32),
        pltpu.VMEM((ZROWS, NHID), jnp.float32),
        pltpu.VMEM_SHARED((ACC_ROWS, NHID), jnp.float32),
        pltpu.SemaphoreType.DMA((2,)),
    ],
)
def _sc_segment_sum(
    ids_hbm, x_hbm, zeros_hbm, out_hbm, ids_v, xbuf, zbuf, acc_sh, sem
):
    cid = lax.axis_index("c")
    sid = lax.axis_index("s")
    wid = sid * NC + cid

    blk0 = wid * BASE_BLK + jnp.minimum(wid, EXTRA)
    nblk = BASE_BLK + jnp.where(wid < EXTRA, 1, 0)

    pltpu.sync_copy(zeros_hbm, zbuf)
    pltpu.sync_copy(zbuf, acc_sh.at[pl.ds(pl.multiple_of(sid * ZROWS, 8), ZROWS)])

    ids0 = pl.multiple_of((blk0 // 8) * 8, 8)
    pltpu.sync_copy(ids_hbm.at[pl.ds(ids0, 32)], ids_v)

    plsc.subcore_barrier()

    def _xsrc(blk):
        row_base = pl.multiple_of(
            jnp.where(blk == NFULL, TAIL_BASE, blk * RB), 8
        )
        return x_hbm.at[pl.ds(row_base, RB)]

    def _stage(j, p):
        pltpu.async_copy(_xsrc(blk0 + j), xbuf.at[p], sem.at[p])

    _stage(0, 0)

    def _block(j, carry):
        p = lax.rem(j, 2)
        blk = blk0 + j

        @pl.when(j + 1 < nblk)
        def _():
            _stage(j + 1, 1 - p)

        pltpu.make_async_copy(_xsrc(blk), xbuf.at[p], sem.at[p]).wait()
        pltpu.sync_copy(xbuf.at[p], acc_sh.at[ids_v.at[blk - ids0]], add=True)
        return carry

    lax.fori_loop(0, nblk, _block, 0)

    plsc.subcore_barrier()

    rows_out = G // NS
    r0 = pl.multiple_of(sid * rows_out, 8)
    pltpu.sync_copy(acc_sh.at[pl.ds(r0, rows_out)], zbuf.at[pl.ds(0, rows_out)])
    pltpu.sync_copy(zbuf.at[pl.ds(0, rows_out)], out_hbm.at[cid].at[pl.ds(r0, rows_out)])


def _tc_finish_body(p_ref, w_ref, b_ref, o_ref):
    pooled = p_ref[0] + p_ref[1]
    o_ref[...] = (
        lax.dot_general(
            pooled,
            w_ref[...],
            (((1,), (1,)), ((), ())),
            preferred_element_type=jnp.float32,
        )
        + b_ref[...]
    )


def kernel(x, edge_attr, node_batch, W, b):
    del edge_attr
    ids = node_batch.astype(jnp.int32)
    tail_ids = jnp.concatenate(
        [jnp.full((RB - TAIL,), PAD_SEG, jnp.int32), ids[NFULL * RB :]]
    )
    ids2d = jnp.concatenate([ids[: NFULL * RB], tail_ids]).reshape(NBLOCKS, RB)
    ids2d = jnp.concatenate(
        [ids2d, jnp.full((IDS_ROWS - NBLOCKS, RB), PAD_SEG, jnp.int32)]
    )
    zeros = jnp.zeros((ZROWS, NHID), jnp.float32)
    partials = _sc_segment_sum(ids2d, x, zeros)
    out = pl.pallas_call(
        _tc_finish_body,
        out_shape=jax.ShapeDtypeStruct((G, NHID), jnp.float32),
    )(partials, W, b.reshape(1, NHID))
    return out


# speedup vs baseline: 6.4516x; 1.0629x over previous
import functools

import jax
import jax.numpy as jnp
from jax import lax
from jax.experimental import pallas as pl
from jax.experimental.pallas import tpu as pltpu
from jax.experimental.pallas import tpu_sc as plsc

N = 100000
NHID = 128
G = 512
NC = 2
NS = 16
NW = NC * NS
RB = 128
NFULL = N // RB
TAIL = N - NFULL * RB
TAIL_BASE = N - RB
TAIL_IDS = NFULL * RB
NBLOCKS = NFULL + 1
BASE_BLK = NBLOCKS // NW
EXTRA = NBLOCKS - BASE_BLK * NW
MAXBLK = BASE_BLK + 1
PAD_SEG = G
ACC_ROWS = 640
ZROWS = ACC_ROWS // NS

_mesh = plsc.VectorSubcoreMesh(
    core_axis_name="c", subcore_axis_name="s", num_cores=NC, num_subcores=NS
)


@functools.partial(
    pl.kernel,
    out_type=jax.ShapeDtypeStruct((NC, G, NHID), jnp.float32),
    mesh=_mesh,
    compiler_params=pltpu.CompilerParams(needs_layout_passes=False),
    scratch_types=[
        pltpu.VMEM((MAXBLK, RB), jnp.int32),
        pltpu.VMEM((2, RB, NHID), jnp.float32),
        pltpu.VMEM((ZROWS, NHID), jnp.float32),
        pltpu.VMEM_SHARED((ACC_ROWS, NHID), jnp.float32),
        pltpu.SemaphoreType.DMA((2,)),
        pltpu.SemaphoreType.DMA,
    ],
)
def _sc_segment_sum(
    ids_hbm, x_hbm, zeros_hbm, pad_hbm, out_hbm, ids_v, xbuf, zbuf, acc_sh, sem, isem
):
    cid = lax.axis_index("c")
    sid = lax.axis_index("s")
    wid = sid * NC + cid

    blk0 = wid * BASE_BLK + jnp.minimum(wid, EXTRA)
    nblk = BASE_BLK + jnp.where(wid < EXTRA, 1, 0)

    pltpu.sync_copy(zeros_hbm, zbuf)
    pltpu.sync_copy(zbuf, acc_sh.at[pl.ds(pl.multiple_of(sid * ZROWS, 8), ZROWS)])

    def _stage_ids(j, carry):
        blk = blk0 + j

        @pl.when(blk < NFULL)
        def _():
            pltpu.async_copy(
                ids_hbm.at[pl.ds(pl.multiple_of(blk * RB, 8), RB)],
                ids_v.at[j],
                isem,
            )

        @pl.when(blk == NFULL)
        def _():
            pltpu.async_copy(pad_hbm, ids_v.at[j], isem)
            pltpu.async_copy(
                ids_hbm.at[pl.ds(TAIL_IDS, TAIL)],
                ids_v.at[j].at[pl.ds(RB - TAIL, TAIL)],
                isem,
            )

        return carry

    lax.fori_loop(0, nblk, _stage_ids, 0)

    def _drain_ids(j, carry):
        blk = blk0 + j

        @pl.when(blk < NFULL)
        def _():
            pltpu.make_async_copy(
                ids_hbm.at[pl.ds(pl.multiple_of(blk * RB, 8), RB)],
                ids_v.at[j],
                isem,
            ).wait()

        @pl.when(blk == NFULL)
        def _():
            pltpu.make_async_copy(pad_hbm, ids_v.at[j], isem).wait()
            pltpu.make_async_copy(
                ids_hbm.at[pl.ds(TAIL_IDS, TAIL)],
                ids_v.at[j].at[pl.ds(RB - TAIL, TAIL)],
                isem,
            ).wait()

        return carry

    lax.fori_loop(0, nblk, _drain_ids, 0)

    plsc.subcore_barrier()

    def _xsrc(blk):
        row_base = pl.multiple_of(
            jnp.where(blk == NFULL, TAIL_BASE, blk * RB), 8
        )
        return x_hbm.at[pl.ds(row_base, RB)]

    def _stage(j, p):
        pltpu.async_copy(_xsrc(blk0 + j), xbuf.at[p], sem.at[p])

    _stage(0, 0)

    def _block(j, carry):
        p = lax.rem(j, 2)
        blk = blk0 + j

        @pl.when(j + 1 < nblk)
        def _():
            _stage(j + 1, 1 - p)

        pltpu.make_async_copy(_xsrc(blk), xbuf.at[p], sem.at[p]).wait()
        pltpu.sync_copy(xbuf.at[p], acc_sh.at[ids_v.at[j]], add=True)
        return carry

    lax.fori_loop(0, nblk, _block, 0)

    plsc.subcore_barrier()

    rows_out = G // NS
    r0 = pl.multiple_of(sid * rows_out, 8)
    pltpu.sync_copy(acc_sh.at[pl.ds(r0, rows_out)], zbuf.at[pl.ds(0, rows_out)])
    pltpu.sync_copy(zbuf.at[pl.ds(0, rows_out)], out_hbm.at[cid].at[pl.ds(r0, rows_out)])


def _tc_finish_body(p_ref, w_ref, b_ref, o_ref):
    pooled = p_ref[0] + p_ref[1]
    o_ref[...] = (
        lax.dot_general(
            pooled,
            w_ref[...],
            (((1,), (1,)), ((), ())),
            preferred_element_type=jnp.float32,
        )
        + b_ref[...]
    )


def kernel(x, edge_attr, node_batch, W, b):
    del edge_attr
    ids = node_batch.astype(jnp.int32)
    zeros = jnp.zeros((ZROWS, NHID), jnp.float32)
    pad_row = jnp.full((RB,), PAD_SEG, jnp.int32)
    partials = _sc_segment_sum(ids, x, zeros, pad_row)
    out = pl.pallas_call(
        _tc_finish_body,
        out_shape=jax.ShapeDtypeStruct((G, NHID), jnp.float32),
    )(partials, W, b.reshape(1, NHID))
    return out


# speedup vs baseline: 6.4552x; 1.0006x over previous
import functools

import jax
import jax.numpy as jnp
from jax import lax
from jax.experimental import pallas as pl
from jax.experimental.pallas import tpu as pltpu
from jax.experimental.pallas import tpu_sc as plsc

N = 100000
NHID = 128
G = 512
NC = 2
NS = 16
NW = NC * NS
RB = 128
NFULL = N // RB
TAIL = N - NFULL * RB
TAIL_BASE = N - RB
TAIL_IDS = NFULL * RB
NBLOCKS = NFULL + 1
BASE_BLK = NBLOCKS // NW
EXTRA = NBLOCKS - BASE_BLK * NW
MAXBLK = BASE_BLK + 1
PAD_SEG = G
ACC_ROWS = 640
ZROWS = ACC_ROWS // NS

_mesh = plsc.VectorSubcoreMesh(
    core_axis_name="c", subcore_axis_name="s", num_cores=NC, num_subcores=NS
)


@functools.partial(
    pl.kernel,
    out_type=jax.ShapeDtypeStruct((NC, G, NHID), jnp.float32),
    mesh=_mesh,
    compiler_params=pltpu.CompilerParams(needs_layout_passes=False),
    scratch_types=[
        pltpu.VMEM((MAXBLK, RB), jnp.int32),
        pltpu.VMEM((2, RB, NHID), jnp.float32),
        pltpu.VMEM((ZROWS, NHID), jnp.float32),
        pltpu.VMEM_SHARED((ACC_ROWS, NHID), jnp.float32),
        pltpu.SemaphoreType.DMA((2,)),
        pltpu.SemaphoreType.DMA,
    ],
)
def _sc_segment_sum(
    ids_hbm, x_hbm, zeros_hbm, pad_hbm, out_hbm, ids_v, xbuf, zbuf, acc_sh, sem, isem
):
    cid = lax.axis_index("c")
    sid = lax.axis_index("s")
    wid = sid * NC + cid

    blk0 = wid * BASE_BLK + jnp.minimum(wid, EXTRA)
    nblk = BASE_BLK + jnp.where(wid < EXTRA, 1, 0)

    pltpu.sync_copy(zeros_hbm, zbuf)
    pltpu.sync_copy(zbuf, acc_sh.at[pl.ds(pl.multiple_of(sid * ZROWS, 8), ZROWS)])

    def _stage_ids(j, carry):
        blk = blk0 + j

        @pl.when(blk < NFULL)
        def _():
            pltpu.async_copy(
                ids_hbm.at[pl.ds(pl.multiple_of(blk * RB, 8), RB)],
                ids_v.at[j],
                isem,
            )

        @pl.when(blk == NFULL)
        def _():
            pltpu.sync_copy(pad_hbm, ids_v.at[j])
            pltpu.async_copy(
                ids_hbm.at[pl.ds(TAIL_IDS, TAIL)],
                ids_v.at[j].at[pl.ds(RB - TAIL, TAIL)],
                isem,
            )

        return carry

    lax.fori_loop(0, nblk, _stage_ids, 0)

    def _drain_ids(j, carry):
        blk = blk0 + j

        @pl.when(blk < NFULL)
        def _():
            pltpu.make_async_copy(
                ids_hbm.at[pl.ds(pl.multiple_of(blk * RB, 8), RB)],
                ids_v.at[j],
                isem,
            ).wait()

        @pl.when(blk == NFULL)
        def _():
            pltpu.make_async_copy(
                ids_hbm.at[pl.ds(TAIL_IDS, TAIL)],
                ids_v.at[j].at[pl.ds(RB - TAIL, TAIL)],
                isem,
            ).wait()

        return carry

    lax.fori_loop(0, nblk, _drain_ids, 0)

    plsc.subcore_barrier()

    def _xsrc(blk):
        row_base = pl.multiple_of(
            jnp.where(blk == NFULL, TAIL_BASE, blk * RB), 8
        )
        return x_hbm.at[pl.ds(row_base, RB)]

    def _stage(j, p):
        pltpu.async_copy(_xsrc(blk0 + j), xbuf.at[p], sem.at[p])

    _stage(0, 0)

    def _block(j, carry):
        p = lax.rem(j, 2)
        blk = blk0 + j

        @pl.when(j + 1 < nblk)
        def _():
            _stage(j + 1, 1 - p)

        pltpu.make_async_copy(_xsrc(blk), xbuf.at[p], sem.at[p]).wait()
        pltpu.sync_copy(xbuf.at[p], acc_sh.at[ids_v.at[j]], add=True)
        return carry

    lax.fori_loop(0, nblk, _block, 0)

    plsc.subcore_barrier()

    rows_out = G // NS
    r0 = pl.multiple_of(sid * rows_out, 8)
    pltpu.sync_copy(acc_sh.at[pl.ds(r0, rows_out)], zbuf.at[pl.ds(0, rows_out)])
    pltpu.sync_copy(zbuf.at[pl.ds(0, rows_out)], out_hbm.at[cid].at[pl.ds(r0, rows_out)])


def _tc_finish_body(p_ref, w_ref, b_ref, o_ref):
    pooled = p_ref[0] + p_ref[1]
    o_ref[...] = (
        lax.dot_general(
            pooled,
            w_ref[...],
            (((1,), (1,)), ((), ())),
            preferred_element_type=jnp.float32,
        )
        + b_ref[...]
    )


def kernel(x, edge_attr, node_batch, W, b):
    del edge_attr
    ids = node_batch.astype(jnp.int32)
    zeros = jnp.zeros((ZROWS, NHID), jnp.float32)
    pad_row = jnp.full((RB,), PAD_SEG, jnp.int32)
    partials = _sc_segment_sum(ids, x, zeros, pad_row)
    out = pl.pallas_call(
        _tc_finish_body,
        out_shape=jax.ShapeDtypeStruct((G, NHID), jnp.float32),
    )(partials, W, b.reshape(1, NHID))
    return out


# speedup vs baseline: 6.5281x; 1.0113x over previous
import functools

import jax
import jax.numpy as jnp
from jax import lax
from jax.experimental import pallas as pl
from jax.experimental.pallas import tpu as pltpu
from jax.experimental.pallas import tpu_sc as plsc

N = 100000
NHID = 128
G = 512
NC = 2
NS = 16
NW = NC * NS
RB = 128
NFULL = N // RB
TAIL = N - NFULL * RB
TAIL_BASE = N - RB
TAIL_IDS = NFULL * RB
NBLOCKS = NFULL + 1
BASE_BLK = NBLOCKS // NW
EXTRA = NBLOCKS - BASE_BLK * NW
MAXBLK = BASE_BLK + 1
PAD_SEG = G
ACC_ROWS = 640
ZROWS = ACC_ROWS // NS
L = 16
NJ = NHID // L
CAP = 64

_mesh = plsc.VectorSubcoreMesh(
    core_axis_name="c", subcore_axis_name="s", num_cores=NC, num_subcores=NS
)


@functools.partial(
    pl.kernel,
    out_type=jax.ShapeDtypeStruct((NC, G, NHID), jnp.float32),
    mesh=_mesh,
    compiler_params=pltpu.CompilerParams(needs_layout_passes=False),
    scratch_types=[
        pltpu.VMEM((MAXBLK, RB), jnp.int32),
        pltpu.VMEM((3, RB, NHID), jnp.float32),
        pltpu.VMEM((ZROWS, NHID), jnp.float32),
        pltpu.VMEM_SHARED((ACC_ROWS, NHID), jnp.float32),
        pltpu.VMEM((CAP, NHID), jnp.float32),
        pltpu.VMEM((8, CAP), jnp.int32),
        pltpu.SemaphoreType.DMA((3,)),
        pltpu.SemaphoreType.DMA,
        pltpu.SemaphoreType.DMA((3,)),
    ],
)
def _sc_segment_sum(
    ids_hbm, x_hbm, zeros_hbm, pad_hbm, out_hbm,
    ids_v, xbuf, zbuf, acc_sh, accl, idx2d, sem, isem, sem_sc
):
    cid = lax.axis_index("c")
    sid = lax.axis_index("s")
    wid = sid * NC + cid

    blk0 = wid * BASE_BLK + jnp.minimum(wid, EXTRA)
    nblk = BASE_BLK + jnp.where(wid < EXTRA, 1, 0)

    pltpu.sync_copy(zeros_hbm, zbuf)
    pltpu.sync_copy(zbuf, acc_sh.at[pl.ds(pl.multiple_of(sid * ZROWS, 8), ZROWS)])

    def _stage_ids(j, carry):
        blk = blk0 + j

        @pl.when(blk < NFULL)
        def _():
            pltpu.async_copy(
                ids_hbm.at[pl.ds(pl.multiple_of(blk * RB, 8), RB)],
                ids_v.at[j],
                isem,
            )

        @pl.when(blk == NFULL)
        def _():
            pltpu.sync_copy(pad_hbm, ids_v.at[j])
            pltpu.async_copy(
                ids_hbm.at[pl.ds(TAIL_IDS, TAIL)],
                ids_v.at[j].at[pl.ds(RB - TAIL, TAIL)],
                isem,
            )

        return carry

    lax.fori_loop(0, nblk, _stage_ids, 0)

    def _drain_ids(j, carry):
        blk = blk0 + j

        @pl.when(blk < NFULL)
        def _():
            pltpu.make_async_copy(
                ids_hbm.at[pl.ds(pl.multiple_of(blk * RB, 8), RB)],
                ids_v.at[j],
                isem,
            ).wait()

        @pl.when(blk == NFULL)
        def _():
            pltpu.make_async_copy(
                ids_hbm.at[pl.ds(TAIL_IDS, TAIL)],
                ids_v.at[j].at[pl.ds(RB - TAIL, TAIL)],
                isem,
            ).wait()

        return carry

    lax.fori_loop(0, nblk, _drain_ids, 0)

    iota = lax.broadcasted_iota(jnp.int32, (L,), 0)
    zero16 = jnp.zeros((L,), jnp.float32)
    seg_lo = jnp.min(ids_v[0, pl.ds(0, L)])
    seg_hi = jnp.max(ids_v[nblk - 1, pl.ds(NHID - L, L)])
    fallback = (seg_hi - seg_lo) >= CAP

    def _zero_accl(g, carry):
        for jj in range(NJ):
            accl[g, pl.ds(L * jj, L)] = zero16
        return carry

    lax.fori_loop(0, CAP, _zero_accl, 0)

    seg_lo16 = jnp.full((L,), seg_lo, jnp.int32)
    for m in range(CAP // L):
        idx2d[0, pl.ds(L * m, L)] = seg_lo16 + iota + L * m

    plsc.subcore_barrier()

    def _xsrc(blk):
        row_base = pl.multiple_of(
            jnp.where(blk == NFULL, TAIL_BASE, blk * RB), 8
        )
        return x_hbm.at[pl.ds(row_base, RB)]

    def _stage(j, p):
        pltpu.async_copy(_xsrc(blk0 + j), xbuf.at[p], sem.at[p])

    def _wait_scatter(j, p):
        pltpu.make_async_copy(
            xbuf.at[p], acc_sh.at[ids_v.at[j]], sem_sc.at[p]
        ).wait()

    _stage(0, 0)

    def _block(j, pend):
        p = lax.rem(j, 3)
        q = lax.rem(j + 1, 3)
        blk = blk0 + j
        pend_q = jnp.where(q == 0, pend[0], jnp.where(q == 1, pend[1], pend[2]))

        @pl.when(pend_q > 0)
        def _():
            _wait_scatter(pend_q - 1, q)

        @pl.when(j + 1 < nblk)
        def _():
            _stage(j + 1, q)

        pltpu.make_async_copy(_xsrc(blk), xbuf.at[p], sem.at[p]).wait()

        single = jnp.min(ids_v[j, pl.ds(0, L)]) == jnp.max(
            ids_v[j, pl.ds(NHID - L, L)]
        )
        use_vpu = jnp.logical_and(single, jnp.logical_not(fallback))

        def _vpu(_):
            def _rows(rr, chains):
                r = rr * 2
                mid = tuple(
                    chains[jj] + xbuf[p, r, pl.ds(L * jj, L)]
                    for jj in range(NJ)
                )
                return tuple(
                    mid[jj] + xbuf[p, r + 1, pl.ds(L * jj, L)]
                    for jj in range(NJ)
                )

            chains = lax.fori_loop(
                0, RB // 2, _rows, tuple(zero16 for _ in range(NJ))
            )
            row = jnp.min(ids_v[j, pl.ds(0, L)]) - seg_lo
            for jj in range(NJ):
                accl[row, pl.ds(L * jj, L)] = (
                    accl[row, pl.ds(L * jj, L)] + chains[jj]
                )
            return 0

        def _stream(_):
            pltpu.make_async_copy(
                xbuf.at[p], acc_sh.at[ids_v.at[j]], sem_sc.at[p]
            ).start(add=True)
            return 0

        lax.cond(use_vpu, _vpu, _stream, 0)
        issued = jnp.where(use_vpu, 0, j + 1).astype(jnp.int32)
        return (
            jnp.where(p == 0, issued, jnp.where(q == 0, jnp.int32(0), pend[0])),
            jnp.where(p == 1, issued, jnp.where(q == 1, jnp.int32(0), pend[1])),
            jnp.where(p == 2, issued, jnp.where(q == 2, jnp.int32(0), pend[2])),
        )

    pend = lax.fori_loop(
        0, nblk, _block, (jnp.int32(0), jnp.int32(0), jnp.int32(0))
    )

    for pp in range(3):
        @pl.when(pend[pp] > 0)
        def _(pp=pp):
            _wait_scatter(pend[pp] - 1, pp)

    @pl.when(jnp.logical_not(fallback))
    def _():
        pltpu.sync_copy(accl, acc_sh.at[idx2d.at[0]], add=True)

    plsc.subcore_barrier()

    rows_out = G // NS
    r0 = pl.multiple_of(sid * rows_out, 8)
    pltpu.sync_copy(acc_sh.at[pl.ds(r0, rows_out)], zbuf.at[pl.ds(0, rows_out)])
    pltpu.sync_copy(zbuf.at[pl.ds(0, rows_out)], out_hbm.at[cid].at[pl.ds(r0, rows_out)])


def _tc_finish_body(p_ref, w_ref, b_ref, o_ref):
    pooled = p_ref[0] + p_ref[1]
    o_ref[...] = (
        lax.dot_general(
            pooled,
            w_ref[...],
            (((1,), (1,)), ((), ())),
            preferred_element_type=jnp.float32,
        )
        + b_ref[...]
    )


def kernel(x, edge_attr, node_batch, W, b):
    del edge_attr
    ids = node_batch.astype(jnp.int32)
    zeros = jnp.zeros((ZROWS, NHID), jnp.float32)
    pad_row = jnp.full((RB,), PAD_SEG, jnp.int32)
    partials = _sc_segment_sum(ids, x, zeros, pad_row)
    out = pl.pallas_call(
        _tc_finish_body,
        out_shape=jax.ShapeDtypeStruct((G, NHID), jnp.float32),
    )(partials, W, b.reshape(1, NHID))
    return out


# speedup vs baseline: 6.7338x; 1.0315x over previous
import functools

import jax
import jax.numpy as jnp
from jax import lax
from jax.experimental import pallas as pl
from jax.experimental.pallas import tpu as pltpu
from jax.experimental.pallas import tpu_sc as plsc

N = 100000
NHID = 128
G = 512
NC = 2
NS = 16
NW = NC * NS
RB = 128
NFULL = N // RB
TAIL = N - NFULL * RB
TAIL_BASE = N - RB
TAIL_IDS = NFULL * RB
NBLOCKS = NFULL + 1
BASE_BLK = NBLOCKS // NW
EXTRA = NBLOCKS - BASE_BLK * NW
MAXBLK = BASE_BLK + 1
PAD_SEG = G
ACC_ROWS = 640
ZROWS = ACC_ROWS // NS
L = 16
NJ = NHID // L
CAP = 64

_mesh = plsc.VectorSubcoreMesh(
    core_axis_name="c", subcore_axis_name="s", num_cores=NC, num_subcores=NS
)


@functools.partial(
    pl.kernel,
    out_type=jax.ShapeDtypeStruct((NC, G, NHID), jnp.float32),
    mesh=_mesh,
    compiler_params=pltpu.CompilerParams(needs_layout_passes=False),
    scratch_types=[
        pltpu.VMEM((MAXBLK, RB), jnp.int32),
        pltpu.VMEM((2, RB, NHID), jnp.float32),
        pltpu.VMEM((ZROWS, NHID), jnp.float32),
        pltpu.VMEM_SHARED((ACC_ROWS, NHID), jnp.float32),
        pltpu.VMEM((CAP, NHID), jnp.float32),
        pltpu.VMEM((8, CAP), jnp.int32),
        pltpu.SemaphoreType.DMA((2,)),
        pltpu.SemaphoreType.DMA,
        pltpu.SemaphoreType.DMA((2,)),
    ],
)
def _sc_segment_sum(
    ids_hbm, x_hbm, zeros_hbm, pad_hbm, out_hbm,
    ids_v, xbuf, zbuf, acc_sh, accl, idx2d, sem, isem, sem_sc
):
    cid = lax.axis_index("c")
    sid = lax.axis_index("s")
    wid = sid * NC + cid

    blk0 = wid * BASE_BLK + jnp.minimum(wid, EXTRA)
    nblk = BASE_BLK + jnp.where(wid < EXTRA, 1, 0)

    pltpu.sync_copy(zeros_hbm, zbuf)
    pltpu.sync_copy(zbuf, acc_sh.at[pl.ds(pl.multiple_of(sid * ZROWS, 8), ZROWS)])

    def _stage_ids(j, carry):
        blk = blk0 + j

        @pl.when(blk < NFULL)
        def _():
            pltpu.async_copy(
                ids_hbm.at[pl.ds(pl.multiple_of(blk * RB, 8), RB)],
                ids_v.at[j],
                isem,
            )

        @pl.when(blk == NFULL)
        def _():
            pltpu.sync_copy(pad_hbm, ids_v.at[j])
            pltpu.async_copy(
                ids_hbm.at[pl.ds(TAIL_IDS, TAIL)],
                ids_v.at[j].at[pl.ds(RB - TAIL, TAIL)],
                isem,
            )

        return carry

    lax.fori_loop(0, nblk, _stage_ids, 0)

    def _drain_ids(j, carry):
        blk = blk0 + j

        @pl.when(blk < NFULL)
        def _():
            pltpu.make_async_copy(
                ids_hbm.at[pl.ds(pl.multiple_of(blk * RB, 8), RB)],
                ids_v.at[j],
                isem,
            ).wait()

        @pl.when(blk == NFULL)
        def _():
            pltpu.make_async_copy(
                ids_hbm.at[pl.ds(TAIL_IDS, TAIL)],
                ids_v.at[j].at[pl.ds(RB - TAIL, TAIL)],
                isem,
            ).wait()

        return carry

    lax.fori_loop(0, nblk, _drain_ids, 0)

    iota = lax.broadcasted_iota(jnp.int32, (L,), 0)
    zero16 = jnp.zeros((L,), jnp.float32)
    seg_lo = jnp.min(ids_v[0, pl.ds(0, L)])
    seg_hi = jnp.max(ids_v[nblk - 1, pl.ds(NHID - L, L)])
    fallback = (seg_hi - seg_lo) >= CAP

    def _zero_accl(g, carry):
        for jj in range(NJ):
            accl[g, pl.ds(L * jj, L)] = zero16
        return carry

    lax.fori_loop(0, CAP, _zero_accl, 0)

    seg_lo16 = jnp.full((L,), seg_lo, jnp.int32)
    for m in range(CAP // L):
        idx2d[0, pl.ds(L * m, L)] = seg_lo16 + iota + L * m

    plsc.subcore_barrier()

    def _xsrc(blk):
        row_base = pl.multiple_of(
            jnp.where(blk == NFULL, TAIL_BASE, blk * RB), 8
        )
        return x_hbm.at[pl.ds(row_base, RB)]

    def _stage(j, p):
        pltpu.async_copy(_xsrc(blk0 + j), xbuf.at[p], sem.at[p])

    def _wait_scatter(j, p):
        pltpu.make_async_copy(
            xbuf.at[p], acc_sh.at[ids_v.at[j]], sem_sc.at[p]
        ).wait()

    _stage(0, 0)

    def _block(j, pend):
        pend0, pend1 = pend
        p = lax.rem(j, 2)
        blk = blk0 + j
        other = jnp.where(p == 0, pend1, pend0)

        @pl.when(other > 0)
        def _():
            _wait_scatter(other - 1, 1 - p)

        @pl.when(j + 1 < nblk)
        def _():
            _stage(j + 1, 1 - p)

        pltpu.make_async_copy(_xsrc(blk), xbuf.at[p], sem.at[p]).wait()

        single = jnp.min(ids_v[j, pl.ds(0, L)]) == jnp.max(
            ids_v[j, pl.ds(NHID - L, L)]
        )
        use_vpu = jnp.logical_and(single, jnp.logical_not(fallback))

        def _vpu(_):
            def _rows(r, chains):
                return tuple(
                    chains[jj] + xbuf[p, r, pl.ds(L * jj, L)]
                    for jj in range(NJ)
                )

            chains = lax.fori_loop(
                0, RB, _rows, tuple(zero16 for _ in range(NJ))
            )
            row = jnp.min(ids_v[j, pl.ds(0, L)]) - seg_lo
            for jj in range(NJ):
                accl[row, pl.ds(L * jj, L)] = (
                    accl[row, pl.ds(L * jj, L)] + chains[jj]
                )
            return 0

        def _stream(_):
            pltpu.make_async_copy(
                xbuf.at[p], acc_sh.at[ids_v.at[j]], sem_sc.at[p]
            ).start(add=True)
            return 0

        lax.cond(use_vpu, _vpu, _stream, 0)
        issued = jnp.where(use_vpu, 0, j + 1).astype(jnp.int32)
        pend0n = jnp.where(p == 0, issued, jnp.int32(0))
        pend1n = jnp.where(p == 0, jnp.int32(0), issued)
        return (pend0n, pend1n)

    pend0, pend1 = lax.fori_loop(
        0, nblk, _block, (jnp.int32(0), jnp.int32(0))
    )

    @pl.when(pend0 > 0)
    def _():
        _wait_scatter(pend0 - 1, 0)

    @pl.when(pend1 > 0)
    def _():
        _wait_scatter(pend1 - 1, 1)

    @pl.when(jnp.logical_not(fallback))
    def _():
        pltpu.sync_copy(accl, acc_sh.at[idx2d.at[0]], add=True)

    plsc.subcore_barrier()

    rows_out = G // NS
    r0 = pl.multiple_of(sid * rows_out, 8)
    pltpu.sync_copy(acc_sh.at[pl.ds(r0, rows_out)], zbuf.at[pl.ds(0, rows_out)])
    pltpu.sync_copy(zbuf.at[pl.ds(0, rows_out)], out_hbm.at[cid].at[pl.ds(r0, rows_out)])


def _tc_finish_body(p_ref, w_ref, b_ref, o_ref):
    pooled = p_ref[0] + p_ref[1]
    o_ref[...] = (
        lax.dot_general(
            pooled,
            w_ref[...],
            (((1,), (1,)), ((), ())),
            preferred_element_type=jnp.float32,
        )
        + b_ref[...]
    )


def kernel(x, edge_attr, node_batch, W, b):
    del edge_attr
    ids = node_batch.astype(jnp.int32)
    zeros = jnp.zeros((ZROWS, NHID), jnp.float32)
    pad_row = jnp.full((RB,), PAD_SEG, jnp.int32)
    partials = _sc_segment_sum(ids, x, zeros, pad_row)
    out = pl.pallas_call(
        _tc_finish_body,
        out_shape=jax.ShapeDtypeStruct((G, NHID), jnp.float32),
    )(partials, W, b.reshape(1, NHID))
    return out
